# baseline (speedup 1.0000x reference)
import jax
import jax.numpy as jnp
from jax.experimental import pallas as pl

_B = 8


def _first_argmax(x, axis):
    m = jnp.max(x, axis=axis, keepdims=True)
    iota = jax.lax.broadcasted_iota(jnp.int32, x.shape, axis)
    big = jnp.int32(x.shape[axis])
    return jnp.min(jnp.where(x == m, iota, big), axis=axis)


def _vote_body(x_ref, o_ref):
    xb = x_ref[...]
    b, l, c = xb.shape
    idx = _first_argmax(xb, 2)
    class_iota3 = jax.lax.broadcasted_iota(jnp.int32, (1, 1, c), 2)
    votes = (idx[:, :, None] == class_iota3).astype(jnp.float32)
    counts = votes.sum(axis=1)
    win = _first_argmax(counts, 1)
    class_iota2 = jax.lax.broadcasted_iota(jnp.int32, (b, c), 1)
    o_ref[...] = (win[:, None] == class_iota2).astype(jnp.float32)


def kernel(x):
    n, l, c = x.shape
    return pl.pallas_call(
        _vote_body,
        grid=(n // _B,),
        in_specs=[pl.BlockSpec((_B, l, c), lambda i: (i, 0, 0))],
        out_specs=pl.BlockSpec((_B, c), lambda i: (i, 0)),
        out_shape=jax.ShapeDtypeStruct((n, c), jnp.float32),
    )(x)
